# baseline (speedup 1.0000x reference)
import functools

import jax
import jax.numpy as jnp
from jax import lax
from jax.experimental import pallas as pl
from jax.experimental.pallas import tpu as pltpu
from jax.experimental.pallas import tpu_sc as plsc

B = 16384
EMB = 32
NFEAT = 26
OUTW = NFEAT * EMB
NC = 2
NS = 16
NW = NC * NS
BPW = B // NW


@functools.partial(
    pl.kernel,
    mesh=plsc.VectorSubcoreMesh(core_axis_name="c", subcore_axis_name="s"),
    out_type=jax.ShapeDtypeStruct((B, OUTW), jnp.bfloat16),
    scratch_types=[
        pltpu.VMEM((NFEAT, BPW), jnp.int32),
        pltpu.VMEM((2, BPW, EMB), jnp.bfloat16),
        pltpu.SemaphoreType.DMA,
        pltpu.SemaphoreType.DMA,
    ],
    compiler_params=pltpu.CompilerParams(use_tc_tiling_on_sc=False),
)
def _lookup_concat(*refs):
    idx_hbm = refs[0]
    tables = refs[1:1 + NFEAT]
    out_hbm = refs[1 + NFEAT]
    idx_v, buf_v, gsem, wsem = refs[2 + NFEAT:]

    wid = lax.axis_index("s") * NC + lax.axis_index("c")
    base = wid * BPW
    pltpu.sync_copy(idx_hbm.at[wid], idx_v)

    gathers = [
        pltpu.make_async_copy(
            tables[f].at[idx_v.at[f]], buf_v.at[f % 2], gsem
        )
        for f in range(NFEAT)
    ]
    writes = [
        pltpu.make_async_copy(
            buf_v.at[f % 2],
            out_hbm.at[pl.ds(base, BPW), pl.ds(f * EMB, EMB)],
            wsem,
        )
        for f in range(NFEAT)
    ]
    gathers[0].start()
    for f in range(NFEAT):
        if f + 1 < NFEAT:
            if f >= 1:
                writes[f - 1].wait()
            gathers[f + 1].start()
        gathers[f].wait()
        writes[f].start()
    writes[NFEAT - 2].wait()
    writes[NFEAT - 1].wait()


def kernel(f00, W_f00, f01, W_f01, f02, W_f02, f03, W_f03, f04, W_f04,
           f05, W_f05, f06, W_f06, f07, W_f07, f08, W_f08, f09, W_f09,
           f10, W_f10, f11, W_f11, f12, W_f12, f13, W_f13, f14, W_f14,
           f15, W_f15, f16, W_f16, f17, W_f17, f18, W_f18, f19, W_f19,
           f20, W_f20, f21, W_f21, f22, W_f22, f23, W_f23, f24, W_f24,
           f25, W_f25):
    idxs = [f00, f01, f02, f03, f04, f05, f06, f07, f08, f09, f10, f11,
            f12, f13, f14, f15, f16, f17, f18, f19, f20, f21, f22, f23,
            f24, f25]
    tables = [W_f00, W_f01, W_f02, W_f03, W_f04, W_f05, W_f06, W_f07,
              W_f08, W_f09, W_f10, W_f11, W_f12, W_f13, W_f14, W_f15,
              W_f16, W_f17, W_f18, W_f19, W_f20, W_f21, W_f22, W_f23,
              W_f24, W_f25]
    tables16 = [w.astype(jnp.bfloat16) for w in tables]
    idx_all = jnp.stack(idxs).reshape(NFEAT, NW, BPW).transpose(1, 0, 2)
    return _lookup_concat(idx_all, *tables16).astype(jnp.float32)


# speedup vs baseline: 1.4796x; 1.4796x over previous
import functools

import jax
import jax.numpy as jnp
from jax import lax
from jax.experimental import pallas as pl
from jax.experimental.pallas import tpu as pltpu
from jax.experimental.pallas import tpu_sc as plsc

B = 16384
EMB = 32
PAD = 128
NFEAT = 26
OUTW = NFEAT * EMB
NC = 2
NS = 16
NW = NC * NS
BPW = B // NW
CH = 256
NCH = BPW // CH
NSTEP = NFEAT * NCH


@functools.partial(
    pl.kernel,
    mesh=plsc.VectorSubcoreMesh(core_axis_name="c", subcore_axis_name="s"),
    out_type=jax.ShapeDtypeStruct((B, OUTW), jnp.float32),
    scratch_types=[
        pltpu.VMEM((NFEAT, BPW), jnp.int32),
        pltpu.VMEM((2, CH, PAD), jnp.float32),
        pltpu.SemaphoreType.DMA,
        pltpu.SemaphoreType.DMA,
    ],
    compiler_params=pltpu.CompilerParams(use_tc_tiling_on_sc=False),
)
def _lookup_concat(*refs):
    idx_hbm = refs[0]
    tables = refs[1:1 + NFEAT]
    out_hbm = refs[1 + NFEAT]
    idx_v, buf_v, gsem, wsem = refs[2 + NFEAT:]

    wid = lax.axis_index("s") * NC + lax.axis_index("c")
    base = wid * BPW
    pltpu.sync_copy(idx_hbm.at[wid], idx_v)

    steps = [(f, h) for f in range(NFEAT) for h in range(NCH)]
    gathers = []
    writes = []
    for s, (f, h) in enumerate(steps):
        p = s % 2
        gathers.append(pltpu.make_async_copy(
            tables[f].at[idx_v.at[f, pl.ds(h * CH, CH)]], buf_v.at[p], gsem
        ))
        writes.append(pltpu.make_async_copy(
            buf_v.at[p, :, pl.ds(0, EMB)],
            out_hbm.at[pl.ds(base + h * CH, CH), pl.ds(f * EMB, EMB)],
            wsem,
        ))
    gathers[0].start()
    for s in range(NSTEP):
        if s + 1 < NSTEP:
            if s >= 1:
                writes[s - 1].wait()
            gathers[s + 1].start()
        gathers[s].wait()
        writes[s].start()
    writes[NSTEP - 2].wait()
    writes[NSTEP - 1].wait()


def kernel(f00, W_f00, f01, W_f01, f02, W_f02, f03, W_f03, f04, W_f04,
           f05, W_f05, f06, W_f06, f07, W_f07, f08, W_f08, f09, W_f09,
           f10, W_f10, f11, W_f11, f12, W_f12, f13, W_f13, f14, W_f14,
           f15, W_f15, f16, W_f16, f17, W_f17, f18, W_f18, f19, W_f19,
           f20, W_f20, f21, W_f21, f22, W_f22, f23, W_f23, f24, W_f24,
           f25, W_f25):
    idxs = [f00, f01, f02, f03, f04, f05, f06, f07, f08, f09, f10, f11,
            f12, f13, f14, f15, f16, f17, f18, f19, f20, f21, f22, f23,
            f24, f25]
    tables = [W_f00, W_f01, W_f02, W_f03, W_f04, W_f05, W_f06, W_f07,
              W_f08, W_f09, W_f10, W_f11, W_f12, W_f13, W_f14, W_f15,
              W_f16, W_f17, W_f18, W_f19, W_f20, W_f21, W_f22, W_f23,
              W_f24, W_f25]
    padded = [jnp.pad(w.T, ((0, PAD - EMB), (0, 0))).T for w in tables]
    idx_all = jnp.stack(idxs).reshape(NFEAT, NW, BPW).transpose(1, 0, 2)
    return _lookup_concat(idx_all, *padded)


# speedup vs baseline: 1.5797x; 1.0676x over previous
import functools

import jax
import jax.numpy as jnp
from jax import lax
from jax.experimental import pallas as pl
from jax.experimental.pallas import tpu as pltpu
from jax.experimental.pallas import tpu_sc as plsc

B = 16384
EMB = 32
NFEAT = 26
OUTW = NFEAT * EMB
NC = 2
NS = 16
NW = NC * NS
BPW = B // NW


@functools.partial(
    pl.kernel,
    mesh=plsc.VectorSubcoreMesh(core_axis_name="c", subcore_axis_name="s"),
    out_type=jax.ShapeDtypeStruct((B, OUTW), jnp.float32),
    scratch_types=[
        pltpu.VMEM((NFEAT, BPW), jnp.int32),
        pltpu.VMEM((2, BPW, EMB), jnp.float32),
        pltpu.SemaphoreType.DMA,
        pltpu.SemaphoreType.DMA,
    ],
    compiler_params=pltpu.CompilerParams(use_tc_tiling_on_sc=False),
)
def _lookup_concat(*refs):
    idx_hbm = refs[0]
    tables = refs[1:1 + NFEAT]
    out_hbm = refs[1 + NFEAT]
    idx_v, buf_v, gsem, wsem = refs[2 + NFEAT:]

    wid = lax.axis_index("s") * NC + lax.axis_index("c")
    base = wid * BPW
    pltpu.sync_copy(idx_hbm.at[wid], idx_v)

    gathers = [
        pltpu.make_async_copy(
            tables[f].at[idx_v.at[f]], buf_v.at[f % 2], gsem
        )
        for f in range(NFEAT)
    ]
    writes = [
        pltpu.make_async_copy(
            buf_v.at[f % 2],
            out_hbm.at[pl.ds(base, BPW), pl.ds(f * EMB, EMB)],
            wsem,
        )
        for f in range(NFEAT)
    ]
    gathers[0].start()
    for f in range(NFEAT):
        if f + 1 < NFEAT:
            if f >= 1:
                writes[f - 1].wait()
            gathers[f + 1].start()
        gathers[f].wait()
        writes[f].start()
    writes[NFEAT - 2].wait()
    writes[NFEAT - 1].wait()


def kernel(f00, W_f00, f01, W_f01, f02, W_f02, f03, W_f03, f04, W_f04,
           f05, W_f05, f06, W_f06, f07, W_f07, f08, W_f08, f09, W_f09,
           f10, W_f10, f11, W_f11, f12, W_f12, f13, W_f13, f14, W_f14,
           f15, W_f15, f16, W_f16, f17, W_f17, f18, W_f18, f19, W_f19,
           f20, W_f20, f21, W_f21, f22, W_f22, f23, W_f23, f24, W_f24,
           f25, W_f25):
    idxs = [f00, f01, f02, f03, f04, f05, f06, f07, f08, f09, f10, f11,
            f12, f13, f14, f15, f16, f17, f18, f19, f20, f21, f22, f23,
            f24, f25]
    tables = [W_f00, W_f01, W_f02, W_f03, W_f04, W_f05, W_f06, W_f07,
              W_f08, W_f09, W_f10, W_f11, W_f12, W_f13, W_f14, W_f15,
              W_f16, W_f17, W_f18, W_f19, W_f20, W_f21, W_f22, W_f23,
              W_f24, W_f25]
    idx_all = jnp.stack(idxs).reshape(NFEAT, NW, BPW).transpose(1, 0, 2)
    return _lookup_concat(idx_all, *tables)


# speedup vs baseline: 1.7995x; 1.1391x over previous
import functools

import jax
import jax.numpy as jnp
from jax import lax
from jax.experimental import pallas as pl
from jax.experimental.pallas import tpu as pltpu
from jax.experimental.pallas import tpu_sc as plsc

B = 16384
EMB = 32
NFEAT = 26
OUTW = NFEAT * EMB
NC = 2
NS = 16
NW = NC * NS
BPW = B // NW


@functools.partial(
    pl.kernel,
    mesh=plsc.VectorSubcoreMesh(core_axis_name="c", subcore_axis_name="s"),
    out_type=jax.ShapeDtypeStruct((OUTW, B), jnp.float32),
    scratch_types=[
        pltpu.VMEM((NFEAT, BPW), jnp.int32),
        pltpu.VMEM((2, EMB, BPW), jnp.float32),
        pltpu.SemaphoreType.DMA,
        pltpu.SemaphoreType.DMA,
    ],
    compiler_params=pltpu.CompilerParams(use_tc_tiling_on_sc=False),
)
def _lookup_concat_t(*refs):
    idx_hbm = refs[0]
    tables_t = refs[1:1 + NFEAT]
    out_hbm = refs[1 + NFEAT]
    idx_v, ebuf, gsem, wsem = refs[2 + NFEAT:]

    wid = lax.axis_index("s") * NC + lax.axis_index("c")
    base = wid * BPW
    pltpu.sync_copy(idx_hbm.at[wid], idx_v)

    writes = [
        pltpu.make_async_copy(
            ebuf.at[f % 2],
            out_hbm.at[pl.ds(f * EMB, EMB), pl.ds(base, BPW)],
            wsem,
        )
        for f in range(NFEAT)
    ]
    for f in range(NFEAT):
        p = f % 2
        if f >= 2:
            writes[f - 2].wait()

        @pl.loop(0, EMB)
        def _g(c, f=f, p=p):
            pltpu.async_copy(
                tables_t[f].at[c].at[idx_v.at[f]], ebuf.at[p, c], gsem
            )

        pltpu.make_async_copy(
            out_hbm.at[pl.ds(0, EMB), pl.ds(0, BPW)], ebuf.at[p], gsem
        ).wait()
        writes[f].start()
    writes[NFEAT - 2].wait()
    writes[NFEAT - 1].wait()


def kernel(f00, W_f00, f01, W_f01, f02, W_f02, f03, W_f03, f04, W_f04,
           f05, W_f05, f06, W_f06, f07, W_f07, f08, W_f08, f09, W_f09,
           f10, W_f10, f11, W_f11, f12, W_f12, f13, W_f13, f14, W_f14,
           f15, W_f15, f16, W_f16, f17, W_f17, f18, W_f18, f19, W_f19,
           f20, W_f20, f21, W_f21, f22, W_f22, f23, W_f23, f24, W_f24,
           f25, W_f25):
    idxs = [f00, f01, f02, f03, f04, f05, f06, f07, f08, f09, f10, f11,
            f12, f13, f14, f15, f16, f17, f18, f19, f20, f21, f22, f23,
            f24, f25]
    tables = [W_f00, W_f01, W_f02, W_f03, W_f04, W_f05, W_f06, W_f07,
              W_f08, W_f09, W_f10, W_f11, W_f12, W_f13, W_f14, W_f15,
              W_f16, W_f17, W_f18, W_f19, W_f20, W_f21, W_f22, W_f23,
              W_f24, W_f25]
    tables_t = [w.T for w in tables]
    idx_all = jnp.stack(idxs).reshape(NFEAT, NW, BPW).transpose(1, 0, 2)
    out_t = _lookup_concat_t(idx_all, *tables_t)
    return out_t.T


# speedup vs baseline: 2.1420x; 1.1903x over previous
import functools

import jax
import jax.numpy as jnp
from jax import lax
from jax.experimental import pallas as pl
from jax.experimental.pallas import tpu as pltpu
from jax.experimental.pallas import tpu_sc as plsc

B = 16384
EMB = 32
NFEAT = 26
NSPLIT = 13
NC = 2
NS = 16
NW = NC * NS
BPW = B // NW


def _make_lookup(nf):
    @functools.partial(
        pl.kernel,
        mesh=plsc.VectorSubcoreMesh(core_axis_name="c", subcore_axis_name="s"),
        out_type=jax.ShapeDtypeStruct((nf * EMB, B), jnp.float32),
        scratch_types=[
            pltpu.VMEM((nf, BPW), jnp.int32),
            pltpu.VMEM((2, EMB, BPW), jnp.float32),
            pltpu.SemaphoreType.DMA,
            pltpu.SemaphoreType.DMA,
        ],
        compiler_params=pltpu.CompilerParams(use_tc_tiling_on_sc=False),
    )
    def _lookup_t(*refs):
        idx_hbm = refs[0]
        tables_t = refs[1:1 + nf]
        out_hbm = refs[1 + nf]
        idx_v, ebuf, gsem, wsem = refs[2 + nf:]

        wid = lax.axis_index("s") * NC + lax.axis_index("c")
        base = wid * BPW
        pltpu.sync_copy(idx_hbm.at[wid], idx_v)

        writes = [
            pltpu.make_async_copy(
                ebuf.at[f % 2],
                out_hbm.at[pl.ds(f * EMB, EMB), pl.ds(base, BPW)],
                wsem,
            )
            for f in range(nf)
        ]
        for f in range(nf):
            p = f % 2
            if f >= 2:
                writes[f - 2].wait()

            @pl.loop(0, EMB)
            def _g(c, f=f, p=p):
                pltpu.async_copy(
                    tables_t[f].at[c].at[idx_v.at[f]], ebuf.at[p, c], gsem
                )

            pltpu.make_async_copy(
                out_hbm.at[pl.ds(0, EMB), pl.ds(0, BPW)], ebuf.at[p], gsem
            ).wait()
            writes[f].start()
        writes[nf - 2].wait()
        writes[nf - 1].wait()

    return _lookup_t


_lookup_half = _make_lookup(NSPLIT)


def kernel(f00, W_f00, f01, W_f01, f02, W_f02, f03, W_f03, f04, W_f04,
           f05, W_f05, f06, W_f06, f07, W_f07, f08, W_f08, f09, W_f09,
           f10, W_f10, f11, W_f11, f12, W_f12, f13, W_f13, f14, W_f14,
           f15, W_f15, f16, W_f16, f17, W_f17, f18, W_f18, f19, W_f19,
           f20, W_f20, f21, W_f21, f22, W_f22, f23, W_f23, f24, W_f24,
           f25, W_f25):
    idxs = [f00, f01, f02, f03, f04, f05, f06, f07, f08, f09, f10, f11,
            f12, f13, f14, f15, f16, f17, f18, f19, f20, f21, f22, f23,
            f24, f25]
    tables = [W_f00, W_f01, W_f02, W_f03, W_f04, W_f05, W_f06, W_f07,
              W_f08, W_f09, W_f10, W_f11, W_f12, W_f13, W_f14, W_f15,
              W_f16, W_f17, W_f18, W_f19, W_f20, W_f21, W_f22, W_f23,
              W_f24, W_f25]
    tables_t = [w.T for w in tables]
    outs = []
    for h in range(NFEAT // NSPLIT):
        sub_idx = idxs[h * NSPLIT:(h + 1) * NSPLIT]
        sub_tab = tables_t[h * NSPLIT:(h + 1) * NSPLIT]
        idx_all = (jnp.stack(sub_idx)
                   .reshape(NSPLIT, NW, BPW).transpose(1, 0, 2))
        outs.append(_lookup_half(idx_all, *sub_tab))
    return jnp.concatenate(outs, axis=0).T


# speedup vs baseline: 2.3714x; 1.1071x over previous
import functools

import jax
import jax.numpy as jnp
from jax import lax
from jax.experimental import pallas as pl
from jax.experimental.pallas import tpu as pltpu
from jax.experimental.pallas import tpu_sc as plsc

B = 16384
EMB = 32
NFEAT = 26
NSPLIT = 13
NC = 2
NS = 16
NW = NC * NS
BPW = B // NW


def _make_lookup(nf):
    @functools.partial(
        pl.kernel,
        mesh=plsc.VectorSubcoreMesh(core_axis_name="c", subcore_axis_name="s"),
        out_type=jax.ShapeDtypeStruct((nf * EMB, B), jnp.float32),
        scratch_types=[
            pltpu.VMEM((nf, BPW), jnp.int32),
            pltpu.VMEM((2, EMB, BPW), jnp.float32),
            pltpu.SemaphoreType.DMA,
            pltpu.SemaphoreType.DMA,
        ],
        compiler_params=pltpu.CompilerParams(use_tc_tiling_on_sc=False),
    )
    def _lookup_t(*refs):
        idx_hbm = refs[0]
        tables_t = refs[1:1 + nf]
        out_hbm = refs[1 + nf]
        idx_v, ebuf, gsem, wsem = refs[2 + nf:]

        wid = lax.axis_index("s") * NC + lax.axis_index("c")
        base = wid * BPW
        pltpu.sync_copy(idx_hbm.at[wid], idx_v)

        writes = [
            pltpu.make_async_copy(
                ebuf.at[f % 2],
                out_hbm.at[pl.ds(f * EMB, EMB), pl.ds(base, BPW)],
                wsem,
            )
            for f in range(nf)
        ]
        for f in range(nf):
            p = f % 2
            if f >= 2:
                writes[f - 2].wait()

            @pl.loop(0, EMB)
            def _g(c, f=f, p=p):
                pltpu.async_copy(
                    tables_t[f].at[c].at[idx_v.at[f]], ebuf.at[p, c], gsem
                )

            pltpu.make_async_copy(
                out_hbm.at[pl.ds(0, EMB), pl.ds(0, BPW)], ebuf.at[p], gsem
            ).wait()
            writes[f].start()
        writes[nf - 2].wait()
        writes[nf - 1].wait()

    return _lookup_t


_lookup_7 = _make_lookup(7)
_lookup_6 = _make_lookup(6)
_SPLITS = [(0, 7, _lookup_7), (7, 7, _lookup_7), (14, 6, _lookup_6),
           (20, 6, _lookup_6)]


def kernel(f00, W_f00, f01, W_f01, f02, W_f02, f03, W_f03, f04, W_f04,
           f05, W_f05, f06, W_f06, f07, W_f07, f08, W_f08, f09, W_f09,
           f10, W_f10, f11, W_f11, f12, W_f12, f13, W_f13, f14, W_f14,
           f15, W_f15, f16, W_f16, f17, W_f17, f18, W_f18, f19, W_f19,
           f20, W_f20, f21, W_f21, f22, W_f22, f23, W_f23, f24, W_f24,
           f25, W_f25):
    idxs = [f00, f01, f02, f03, f04, f05, f06, f07, f08, f09, f10, f11,
            f12, f13, f14, f15, f16, f17, f18, f19, f20, f21, f22, f23,
            f24, f25]
    tables = [W_f00, W_f01, W_f02, W_f03, W_f04, W_f05, W_f06, W_f07,
              W_f08, W_f09, W_f10, W_f11, W_f12, W_f13, W_f14, W_f15,
              W_f16, W_f17, W_f18, W_f19, W_f20, W_f21, W_f22, W_f23,
              W_f24, W_f25]
    tables_t = [w.T for w in tables]
    outs = []
    for start, nf, fn in _SPLITS:
        sub_idx = idxs[start:start + nf]
        sub_tab = tables_t[start:start + nf]
        idx_all = (jnp.stack(sub_idx)
                   .reshape(nf, NW, BPW).transpose(1, 0, 2))
        outs.append(fn(idx_all, *sub_tab))
    return jnp.concatenate(outs, axis=0).T


# speedup vs baseline: 2.4587x; 1.0368x over previous
import functools

import jax
import jax.numpy as jnp
from jax import lax
from jax.experimental import pallas as pl
from jax.experimental.pallas import tpu as pltpu
from jax.experimental.pallas import tpu_sc as plsc

B = 16384
EMB = 32
NFEAT = 26
NSPLIT = 13
NC = 2
NS = 16
NW = NC * NS
BPW = B // NW


def _make_lookup(nf):
    @functools.partial(
        pl.kernel,
        mesh=plsc.VectorSubcoreMesh(core_axis_name="c", subcore_axis_name="s"),
        out_type=jax.ShapeDtypeStruct((nf * EMB, B), jnp.float32),
        scratch_types=[
            pltpu.VMEM((nf, BPW), jnp.int32),
            pltpu.VMEM((2, EMB, BPW), jnp.float32),
            pltpu.SemaphoreType.DMA,
            pltpu.SemaphoreType.DMA,
        ],
        compiler_params=pltpu.CompilerParams(use_tc_tiling_on_sc=False),
    )
    def _lookup_t(*refs):
        idx_hbm = refs[0]
        tables_t = refs[1:1 + nf]
        out_hbm = refs[1 + nf]
        idx_v, ebuf, gsem, wsem = refs[2 + nf:]

        wid = lax.axis_index("s") * NC + lax.axis_index("c")
        base = wid * BPW
        pltpu.sync_copy(idx_hbm.at[wid], idx_v)

        writes = [
            pltpu.make_async_copy(
                ebuf.at[f % 2],
                out_hbm.at[pl.ds(f * EMB, EMB), pl.ds(base, BPW)],
                wsem,
            )
            for f in range(nf)
        ]
        for f in range(nf):
            p = f % 2
            if f >= 2:
                writes[f - 2].wait()

            @pl.loop(0, EMB)
            def _g(c, f=f, p=p):
                pltpu.async_copy(
                    tables_t[f].at[c].at[idx_v.at[f]], ebuf.at[p, c], gsem
                )

            pltpu.make_async_copy(
                out_hbm.at[pl.ds(0, EMB), pl.ds(0, BPW)], ebuf.at[p], gsem
            ).wait()
            writes[f].start()
        writes[nf - 2].wait()
        writes[nf - 1].wait()

    return _lookup_t


_lookup_5 = _make_lookup(5)
_lookup_4 = _make_lookup(4)
_SPLITS = [(0, 5, _lookup_5), (5, 5, _lookup_5), (10, 4, _lookup_4),
           (14, 4, _lookup_4), (18, 4, _lookup_4), (22, 4, _lookup_4)]


def kernel(f00, W_f00, f01, W_f01, f02, W_f02, f03, W_f03, f04, W_f04,
           f05, W_f05, f06, W_f06, f07, W_f07, f08, W_f08, f09, W_f09,
           f10, W_f10, f11, W_f11, f12, W_f12, f13, W_f13, f14, W_f14,
           f15, W_f15, f16, W_f16, f17, W_f17, f18, W_f18, f19, W_f19,
           f20, W_f20, f21, W_f21, f22, W_f22, f23, W_f23, f24, W_f24,
           f25, W_f25):
    idxs = [f00, f01, f02, f03, f04, f05, f06, f07, f08, f09, f10, f11,
            f12, f13, f14, f15, f16, f17, f18, f19, f20, f21, f22, f23,
            f24, f25]
    tables = [W_f00, W_f01, W_f02, W_f03, W_f04, W_f05, W_f06, W_f07,
              W_f08, W_f09, W_f10, W_f11, W_f12, W_f13, W_f14, W_f15,
              W_f16, W_f17, W_f18, W_f19, W_f20, W_f21, W_f22, W_f23,
              W_f24, W_f25]
    tables_t = [w.T for w in tables]
    outs = []
    for start, nf, fn in _SPLITS:
        sub_idx = idxs[start:start + nf]
        sub_tab = tables_t[start:start + nf]
        idx_all = (jnp.stack(sub_idx)
                   .reshape(nf, NW, BPW).transpose(1, 0, 2))
        outs.append(fn(idx_all, *sub_tab))
    return jnp.concatenate(outs, axis=0).T


# speedup vs baseline: 2.4678x; 1.0037x over previous
import functools

import jax
import jax.numpy as jnp
from jax import lax
from jax.experimental import pallas as pl
from jax.experimental.pallas import tpu as pltpu
from jax.experimental.pallas import tpu_sc as plsc

B = 16384
EMB = 32
NFEAT = 26
NSPLIT = 13
NC = 2
NS = 16
NW = NC * NS
BPW = B // NW


def _make_lookup(nf):
    @functools.partial(
        pl.kernel,
        mesh=plsc.VectorSubcoreMesh(core_axis_name="c", subcore_axis_name="s"),
        out_type=jax.ShapeDtypeStruct((nf * EMB, B), jnp.float32),
        scratch_types=[
            pltpu.VMEM((nf, BPW), jnp.int32),
            pltpu.VMEM((2, EMB, BPW), jnp.float32),
            pltpu.SemaphoreType.DMA,
            pltpu.SemaphoreType.DMA,
        ],
        compiler_params=pltpu.CompilerParams(use_tc_tiling_on_sc=False),
    )
    def _lookup_t(*refs):
        idx_hbm = refs[0]
        tables_t = refs[1:1 + nf]
        out_hbm = refs[1 + nf]
        idx_v, ebuf, gsem, wsem = refs[2 + nf:]

        wid = lax.axis_index("s") * NC + lax.axis_index("c")
        base = wid * BPW
        pltpu.sync_copy(idx_hbm.at[wid], idx_v)

        writes = [
            pltpu.make_async_copy(
                ebuf.at[f % 2],
                out_hbm.at[pl.ds(f * EMB, EMB), pl.ds(base, BPW)],
                wsem,
            )
            for f in range(nf)
        ]
        for f in range(nf):
            p = f % 2
            if f >= 2:
                writes[f - 2].wait()

            @pl.loop(0, EMB)
            def _g(c, f=f, p=p):
                pltpu.async_copy(
                    tables_t[f].at[c].at[idx_v.at[f]], ebuf.at[p, c], gsem
                )

            pltpu.make_async_copy(
                out_hbm.at[pl.ds(0, EMB), pl.ds(0, BPW)], ebuf.at[p], gsem
            ).wait()
            writes[f].start()
        writes[nf - 2].wait()
        writes[nf - 1].wait()

    return _lookup_t


_lookup_4 = _make_lookup(4)
_lookup_3 = _make_lookup(3)
_SPLITS = [(0, 4, _lookup_4), (4, 4, _lookup_4), (8, 3, _lookup_3),
           (11, 3, _lookup_3), (14, 3, _lookup_3), (17, 3, _lookup_3),
           (20, 3, _lookup_3), (23, 3, _lookup_3)]


def kernel(f00, W_f00, f01, W_f01, f02, W_f02, f03, W_f03, f04, W_f04,
           f05, W_f05, f06, W_f06, f07, W_f07, f08, W_f08, f09, W_f09,
           f10, W_f10, f11, W_f11, f12, W_f12, f13, W_f13, f14, W_f14,
           f15, W_f15, f16, W_f16, f17, W_f17, f18, W_f18, f19, W_f19,
           f20, W_f20, f21, W_f21, f22, W_f22, f23, W_f23, f24, W_f24,
           f25, W_f25):
    idxs = [f00, f01, f02, f03, f04, f05, f06, f07, f08, f09, f10, f11,
            f12, f13, f14, f15, f16, f17, f18, f19, f20, f21, f22, f23,
            f24, f25]
    tables = [W_f00, W_f01, W_f02, W_f03, W_f04, W_f05, W_f06, W_f07,
              W_f08, W_f09, W_f10, W_f11, W_f12, W_f13, W_f14, W_f15,
              W_f16, W_f17, W_f18, W_f19, W_f20, W_f21, W_f22, W_f23,
              W_f24, W_f25]
    tables_t = [w.T for w in tables]
    outs = []
    for start, nf, fn in _SPLITS:
        sub_idx = idxs[start:start + nf]
        sub_tab = tables_t[start:start + nf]
        idx_all = (jnp.stack(sub_idx)
                   .reshape(nf, NW, BPW).transpose(1, 0, 2))
        outs.append(fn(idx_all, *sub_tab))
    return jnp.concatenate(outs, axis=0).T
